# baseline (speedup 1.0000x reference)
import functools

import jax
import jax.numpy as jnp
from jax import lax
from jax.experimental import pallas as pl
from jax.experimental.pallas import tpu as pltpu
from jax.experimental.pallas import tpu_sc as plsc

_LANES = 16
_CHUNK = 128


def _make_sc_gather(B, L, F, NC, NI, num_workers):
    rows_per_w = B // num_workers
    elems_per_w = rows_per_w * L
    chunks_per_w = elems_per_w // _CHUNK

    mesh = plsc.VectorSubcoreMesh(core_axis_name="c", subcore_axis_name="s")

    @functools.partial(
        pl.kernel,
        out_type=[
            jax.ShapeDtypeStruct((B, F), jnp.float32),
            jax.ShapeDtypeStruct((num_workers, chunks_per_w, _CHUNK),
                                 jnp.float32),
        ],
        mesh=mesh,
        scratch_types=[
            pltpu.VMEM((rows_per_w,), jnp.int32),
            pltpu.VMEM((rows_per_w, F), jnp.float32),
            pltpu.VMEM((chunks_per_w, _CHUNK), jnp.int32),
            pltpu.VMEM((chunks_per_w, _CHUNK), jnp.int32),
            pltpu.VMEM((chunks_per_w, _CHUNK), jnp.float32),
            pltpu.VMEM((chunks_per_w, _CHUNK), jnp.float32),
            pltpu.VMEM((NI,), jnp.float32),
            pltpu.SemaphoreType.DMA,
            pltpu.SemaphoreType.DMA,
        ],
        compiler_params=pltpu.CompilerParams(
            needs_layout_passes=False, use_tc_tiling_on_sc=False),
    )
    def sc_gather(choosers_hbm, fidx_hbm, items_hbm, thetas_hbm, inter_hbm,
                  gint_hbm, theta_out_hbm, inter_out_hbm,
                  cid_v, rows_v, fidx_v, items_v, gath_v, res_v, gint_v,
                  sem, sem2):
        wid = lax.axis_index("s") * 2 + lax.axis_index("c")
        rbase = wid * rows_per_w

        pltpu.sync_copy(choosers_hbm.at[pl.ds(rbase, rows_per_w)], cid_v)
        theta_cp = pltpu.async_copy(thetas_hbm.at[cid_v], rows_v, sem2)

        pltpu.sync_copy(fidx_hbm.at[wid], fidx_v)
        pltpu.sync_copy(items_hbm.at[wid], items_v)
        pltpu.sync_copy(gint_hbm, gint_v)

        group = 10
        for g in range(chunks_per_w // group):
            cps = [
                pltpu.async_copy(
                    inter_hbm.at[fidx_v.at[g * group + j]],
                    gath_v.at[g * group + j], sem)
                for j in range(group)
            ]
            for cp in cps:
                cp.wait()

        theta_cp.wait()
        pltpu.sync_copy(rows_v, theta_out_hbm.at[pl.ds(rbase, rows_per_w)])

        def body(i, carry):
            r = i // (_CHUNK // _LANES)
            j = i % (_CHUNK // _LANES)
            it16 = items_v[r, pl.ds(j * _LANES, _LANES)]
            g16 = plsc.load_gather(gint_v, [it16])
            res_v[r, pl.ds(j * _LANES, _LANES)] = (
                gath_v[r, pl.ds(j * _LANES, _LANES)] + g16)
            return carry

        lax.fori_loop(0, chunks_per_w * (_CHUNK // _LANES), body, 0)
        pltpu.sync_copy(res_v, inter_out_hbm.at[wid])

    return sc_gather


def _tc_body(feat_ref, theta_ref, gt_ref, inter_ref, sz_ref, out_ref):
    w = theta_ref[...] + gt_ref[...][None, :]
    u = jnp.sum(feat_ref[...] * w[:, None, :], axis=-1)
    u = u + inter_ref[...]
    iota = lax.broadcasted_iota(jnp.int32, u.shape, 1)
    u = jnp.where(iota >= sz_ref[...], -jnp.inf, u)
    m = jnp.max(u, axis=1, keepdims=True)
    e = jnp.exp(u - m)
    out_ref[...] = (u - m) - jnp.log(jnp.sum(e, axis=1, keepdims=True))


def kernel(choice_set_features, choice_set_sizes, choosers, choice_sets,
           thetas, global_theta, intercepts, global_intercept):
    B, L, F = choice_set_features.shape
    NC, NI = intercepts.shape

    choosers = choosers.astype(jnp.int32)
    items = choice_sets.astype(jnp.int32)
    fidx = choosers[:, None] * NI + items
    nw = 32
    fidx3d = fidx.reshape(nw, -1, _CHUNK)
    items3d = items.reshape(nw, -1, _CHUNK)

    sc_gather = _make_sc_gather(B, L, F, NC, NI, num_workers=nw)
    per_theta, inter = sc_gather(
        choosers, fidx3d, items3d, thetas, intercepts.reshape(-1),
        global_intercept)
    inter2d = inter.reshape(B, L)

    BB = 256
    out = pl.pallas_call(
        _tc_body,
        grid=(B // BB,),
        in_specs=[
            pl.BlockSpec((BB, L, F), lambda i: (i, 0, 0)),
            pl.BlockSpec((BB, F), lambda i: (i, 0)),
            pl.BlockSpec((F,), lambda i: (0,)),
            pl.BlockSpec((BB, L), lambda i: (i, 0)),
            pl.BlockSpec((BB, 1), lambda i: (i, 0)),
        ],
        out_specs=pl.BlockSpec((BB, L), lambda i: (i, 0)),
        out_shape=jax.ShapeDtypeStruct((B, L), jnp.float32),
    )(choice_set_features, per_theta, global_theta, inter2d,
      choice_set_sizes.astype(jnp.int32).reshape(B, 1))
    return out


# speedup vs baseline: 3.4747x; 3.4747x over previous
import functools

import jax
import jax.numpy as jnp
from jax import lax
from jax.experimental import pallas as pl
from jax.experimental.pallas import tpu as pltpu
from jax.experimental.pallas import tpu_sc as plsc

_LANES = 16
_CHUNK = 128


def _make_sc_gather(B, L, F, NC, NI, num_workers):
    rows_per_w = B // num_workers
    chunks_per_w = (rows_per_w * L) // _CHUNK

    mesh = plsc.VectorSubcoreMesh(core_axis_name="c", subcore_axis_name="s")

    @functools.partial(
        pl.kernel,
        out_type=[
            jax.ShapeDtypeStruct((B, F), jnp.float32),
            jax.ShapeDtypeStruct((L, B), jnp.float32),
        ],
        mesh=mesh,
        scratch_types=[
            pltpu.VMEM((rows_per_w,), jnp.int32),
            pltpu.VMEM((rows_per_w, F), jnp.float32),
            pltpu.VMEM((chunks_per_w, _CHUNK), jnp.int32),
            pltpu.VMEM((chunks_per_w, _CHUNK), jnp.int32),
            pltpu.VMEM((chunks_per_w, _CHUNK), jnp.float32),
            pltpu.VMEM((chunks_per_w, _CHUNK), jnp.float32),
            pltpu.VMEM((NI,), jnp.float32),
            pltpu.SemaphoreType.DMA,
            pltpu.SemaphoreType.DMA,
        ],
        compiler_params=pltpu.CompilerParams(
            needs_layout_passes=False, use_tc_tiling_on_sc=False),
    )
    def sc_gather(choosers_hbm, fidx_hbm, items_hbm, thetas_hbm, inter_hbm,
                  gint_hbm, theta_out_hbm, inter_out_hbm,
                  cid_v, rows_v, fidx_v, items_v, gath_v, res_v, gint_v,
                  sem, sem2):
        wid = lax.axis_index("s") * 2 + lax.axis_index("c")
        rbase = wid * rows_per_w

        pltpu.sync_copy(choosers_hbm.at[pl.ds(rbase, rows_per_w)], cid_v)
        theta_cp = pltpu.async_copy(thetas_hbm.at[cid_v], rows_v, sem2)

        pltpu.sync_copy(fidx_hbm.at[wid], fidx_v)
        pltpu.sync_copy(items_hbm.at[wid], items_v)
        pltpu.sync_copy(gint_hbm, gint_v)

        group = 10
        for g in range(chunks_per_w // group):
            cps = [
                pltpu.async_copy(
                    inter_hbm.at[fidx_v.at[g * group + j]],
                    gath_v.at[g * group + j], sem)
                for j in range(group)
            ]
            for cp in cps:
                cp.wait()

        theta_cp.wait()
        pltpu.sync_copy(rows_v, theta_out_hbm.at[pl.ds(rbase, rows_per_w)])

        def body(i, carry):
            r = i // (_CHUNK // _LANES)
            j = i % (_CHUNK // _LANES)
            it16 = items_v[r, pl.ds(j * _LANES, _LANES)]
            g16 = plsc.load_gather(gint_v, [it16])
            res_v[r, pl.ds(j * _LANES, _LANES)] = (
                gath_v[r, pl.ds(j * _LANES, _LANES)] + g16)
            return carry

        lax.fori_loop(0, chunks_per_w * (_CHUNK // _LANES), body, 0)
        pltpu.sync_copy(res_v, inter_out_hbm.at[:, pl.ds(rbase, rows_per_w)])

    return sc_gather


def _tc_body(feat_ref, thetaT_ref, inter_ref, sz_ref, out_ref):
    w = thetaT_ref[...]
    u = jnp.sum(feat_ref[...] * w[None, :, :], axis=1)
    u = u + inter_ref[...]
    iota = lax.broadcasted_iota(jnp.int32, u.shape, 0)
    u = jnp.where(iota >= sz_ref[...], -jnp.inf, u)
    m = jnp.max(u, axis=0, keepdims=True)
    e = jnp.exp(u - m)
    out_ref[...] = (u - m) - jnp.log(jnp.sum(e, axis=0, keepdims=True))


def kernel(choice_set_features, choice_set_sizes, choosers, choice_sets,
           thetas, global_theta, intercepts, global_intercept):
    B, L, F = choice_set_features.shape
    NC, NI = intercepts.shape

    choosers = choosers.astype(jnp.int32)
    items = choice_sets.astype(jnp.int32)
    nw = 32
    interT_flat = jnp.transpose(intercepts).reshape(-1)
    fidx = items * NC + choosers[:, None]
    fidxT3 = fidx.T.reshape(L, nw, B // nw).transpose(1, 0, 2)
    itemsT3 = items.T.reshape(L, nw, B // nw).transpose(1, 0, 2)

    sc_gather = _make_sc_gather(B, L, F, NC, NI, num_workers=nw)
    per_theta, interT = sc_gather(
        choosers, fidxT3, itemsT3, thetas, interT_flat, global_intercept)

    featT = jnp.transpose(choice_set_features, (1, 2, 0))
    wT = (per_theta + global_theta[None, :]).T

    BB = 512
    outT = pl.pallas_call(
        _tc_body,
        grid=(B // BB,),
        in_specs=[
            pl.BlockSpec((L, F, BB), lambda i: (0, 0, i)),
            pl.BlockSpec((F, BB), lambda i: (0, i)),
            pl.BlockSpec((L, BB), lambda i: (0, i)),
            pl.BlockSpec((1, BB), lambda i: (0, i)),
        ],
        out_specs=pl.BlockSpec((L, BB), lambda i: (0, i)),
        out_shape=jax.ShapeDtypeStruct((L, B), jnp.float32),
    )(featT, wT, interT, choice_set_sizes.astype(jnp.int32).reshape(1, B))
    return outT.T
